# baseline (speedup 1.0000x reference)
import functools
import math

import jax
import jax.numpy as jnp
from jax import lax
from jax.experimental import pallas as pl
from jax.experimental.pallas import tpu as pltpu
from jax.experimental.pallas import tpu_sc as plsc

_NC = 2
_NS = 16
_NW = _NC * _NS
_L = 16

_B = 16384
_D = 32
_CAP = 8192
_DUMMY = _B
_STAGE_ROWS = _B + _NW * _L
_RING = 4

def _cfg(n):
    tcols = (n + 127) // 128
    cpt = (tcols + _NW - 1) // _NW
    return n, tcols * 128, cpt
_UN, _UPAD, _UCPT = _cfg(1000000)
_PN, _PPAD, _PCPT = _cfg(100000)
_TN, _TPAD, _TCPT = _cfg(1000)
_UW, _UNCH = 640, 49
_PW, _PNCH = 640, 5
_TW, _TNCH = 128, 1


def _stream_table(wid, idx_v, wT_hbm, stage_hbm, band_v, cand_c, cand_b,
                  c2_c, c2_b, rows_v, bidx_v, sems, ssem, cfg):
    n, pad, cpt, w, nch = cfg
    span = cpt * 128
    nom_lo = wid * span
    nom_hi = jnp.minimum(nom_lo + span, n)
    lo_s = jnp.minimum(nom_lo, pad - span)

    def scan_tile(g, tot):
        v = idx_v[pl.ds(g * _L, _L)]
        m = jnp.logical_and(v >= nom_lo, v < nom_hi)
        b = lax.iota(jnp.int32, _L) + g * _L
        plsc.store_compressed(cand_c.at[pl.ds(tot, _L)], v, mask=m)
        plsc.store_compressed(cand_b.at[pl.ds(tot, _L)], b, mask=m)
        return tot + plsc.all_reduce_population_count(m)[0]
    ntile = lax.fori_loop(0, _B // _L, scan_tile, jnp.zeros((), jnp.int32))
    ntile = jnp.minimum(ntile, _CAP)

    def fire(k, slot):
        s_k = lo_s + k * w
        return pltpu.async_copy(wT_hbm.at[:, pl.ds(s_k, w)],
                                band_v.at[slot], sems[slot])

    def process(k, slot):
        s_k = lo_s + k * w
        band = band_v.at[slot]
        def scan_chunk(g, tot):
            v = cand_c[pl.ds(g * _L, _L)]
            bb = cand_b[pl.ds(g * _L, _L)]
            pos_ok = (lax.iota(jnp.int32, _L) + g * _L) < ntile
            m = (v >= s_k) & (v < s_k + w) & pos_ok
            plsc.store_compressed(c2_c.at[pl.ds(tot, _L)], v, mask=m)
            plsc.store_compressed(c2_b.at[pl.ds(tot, _L)], bb, mask=m)
            return tot + plsc.all_reduce_population_count(m)[0]
        ng = lax.div(ntile + (_L - 1), _L)
        nc = lax.fori_loop(0, ng, scan_chunk, jnp.zeros((), jnp.int32))
        nc = jnp.minimum(nc, _CAP)

        rowid = lax.iota(jnp.int32, _L)
        def batch(bi, tot):
            rslot = lax.rem(bi, _RING)
            pos = bi * _L
            cols = jnp.clip(c2_c[pl.ds(pos, _L)] - s_k, 0, w - 1)
            bs = c2_b[pl.ds(pos, _L)]
            valid = (rowid + pos) < nc
            bfin = jnp.where(valid, bs, _DUMMY + wid * _L + rowid)

            @pl.when(bi >= _RING)
            def _():
                pltpu.make_async_copy(rows_v.at[0],
                                      stage_hbm.at[bidx_v.at[0]], ssem).wait()

            for j in range(_L):
                cj = jnp.full((_L,), cols[j], jnp.int32)
                r0 = plsc.load_gather(band, [rowid, cj])
                r1 = plsc.load_gather(band, [rowid + _L, cj])
                rows_v[rslot, j, pl.ds(0, _L)] = r0
                rows_v[rslot, j, pl.ds(_L, _L)] = r1
            bidx_v[rslot, pl.ds(0, _L)] = bfin
            pltpu.async_copy(rows_v.at[rslot],
                             stage_hbm.at[bidx_v.at[rslot]], ssem)
            return tot
        nb = lax.div(nc + (_L - 1), _L)
        lax.fori_loop(0, nb, batch, jnp.zeros((), jnp.int32))
        def drain(r, _):
            @pl.when(r < jnp.minimum(nb, _RING))
            def _():
                pltpu.make_async_copy(rows_v.at[0], stage_hbm.at[bidx_v.at[0]],
                                      ssem).wait()
            return 0
        lax.fori_loop(0, _RING, drain, 0)

    cps = [fire(0, 0)]
    def ring(i, _):
        k0 = i * 2
        fire(k0 + 1, 1)
        _wait(k0, 0)
        process(k0, 0)
        fire(k0 + 2, 0)
        _wait(k0 + 1, 1)
        process(k0 + 1, 1)
        return 0

    def _wait(k, slot):
        pltpu.make_async_copy(wT_hbm.at[:, pl.ds(lo_s, w)],
                              band_v.at[slot], sems[slot]).wait()

    if nch == 1:
        _wait(0, 0)
        process(0, 0)
    else:
        lax.fori_loop(0, (nch - 1) // 2, ring, 0)
        _wait(nch - 1, 0)
        process(nch - 1, 0)


def _a_body(users_hbm, pastors_hbm, traits_hbm, uwT_hbm, pwT_hbm, twT_hbm,
            su_hbm, sp_hbm, st_hbm, idx_v, band_v, tband_v,
            cand_c, cand_b, c2_c, c2_b, rows_v, bidx_v, sem0, sem1, ssem):
    wid = lax.axis_index("s") * _NC + lax.axis_index("c")
    sems = (sem0, sem1)

    pltpu.sync_copy(users_hbm.at[pl.ds(0, _B)], idx_v)
    _stream_table(wid, idx_v, uwT_hbm, su_hbm, band_v, cand_c, cand_b,
                  c2_c, c2_b, rows_v, bidx_v, sems, ssem,
                  (_UN, _UPAD, _UCPT, _UW, _UNCH))
    pltpu.sync_copy(pastors_hbm.at[pl.ds(0, _B)], idx_v)
    _stream_table(wid, idx_v, pwT_hbm, sp_hbm, band_v, cand_c, cand_b,
                  c2_c, c2_b, rows_v, bidx_v, sems, ssem,
                  (_PN, _PPAD, _PCPT, _PW, _PNCH))
    pltpu.sync_copy(traits_hbm.at[pl.ds(0, _B)], idx_v)
    _stream_table(wid, idx_v, twT_hbm, st_hbm, tband_v, cand_c, cand_b,
                  c2_c, c2_b, rows_v, bidx_v, sems, ssem,
                  (_TN, _TPAD, _TCPT, _TW, _TNCH))


_phase_a = functools.partial(
    pl.kernel,
    mesh=plsc.VectorSubcoreMesh(core_axis_name="c", subcore_axis_name="s"),
    out_type=(
        jax.ShapeDtypeStruct((_STAGE_ROWS, 128), jnp.float32),
        jax.ShapeDtypeStruct((_STAGE_ROWS, 128), jnp.float32),
        jax.ShapeDtypeStruct((_STAGE_ROWS, 128), jnp.float32),
    ),
    compiler_params=pltpu.CompilerParams(
        needs_layout_passes=False, use_tc_tiling_on_sc=True),
    scratch_types=[
        pltpu.VMEM((_B,), jnp.int32),
        pltpu.VMEM((2, 32, _UW), jnp.float32),
        pltpu.VMEM((2, 32, _TW), jnp.float32),
        pltpu.VMEM((_CAP,), jnp.int32),
        pltpu.VMEM((_CAP,), jnp.int32),
        pltpu.VMEM((_CAP,), jnp.int32),
        pltpu.VMEM((_CAP,), jnp.int32),
        pltpu.VMEM((_RING, _L, 128), jnp.float32),
        pltpu.VMEM((_RING, _L), jnp.int32),
        pltpu.SemaphoreType.DMA,
        pltpu.SemaphoreType.DMA,
        pltpu.SemaphoreType.DMA,
    ],
)(_a_body)


_BPW = _B // _NW
_BCH = 128


def _b_body(su_hbm, sp_hbm, st_hbm, out_hbm, ub_v, pb_v, tb_v, out_v,
            sem0, sem1):
    wid = lax.axis_index("s") * _NC + lax.axis_index("c")
    b0 = wid * _BPW
    sems = (sem0, sem1)
    nch = _BPW // _BCH

    def fire(j, slot):
        src = pl.ds(b0 + j * _BCH, _BCH)
        return [pltpu.async_copy(su_hbm.at[src], ub_v.at[slot], sems[slot]),
                pltpu.async_copy(sp_hbm.at[src], pb_v.at[slot], sems[slot]),
                pltpu.async_copy(st_hbm.at[src], tb_v.at[slot], sems[slot])]

    inv = 1.0 / math.sqrt(_D)
    lane = lax.iota(jnp.int32, _L)
    pend = {0: fire(0, 0)}
    for j in range(nch):
        slot = j % 2
        if j + 1 < nch:
            pend[(j + 1) % 2] = fire(j + 1, (j + 1) % 2)
        for c in pend.pop(slot):
            c.wait()
        ub = ub_v.at[slot]
        pb = pb_v.at[slot]
        tb = tb_v.at[slot]

        def group(g, _):
            acc = jnp.zeros((_L,), jnp.float32)
            c0 = g * _L
            for k in range(_L):
                i = c0 + k
                u0 = ub[i, pl.ds(0, _L)]
                u1 = ub[i, pl.ds(_L, _L)]
                v0 = pb[i, pl.ds(0, _L)] + tb[i, pl.ds(0, _L)]
                v1 = pb[i, pl.ds(_L, _L)] + tb[i, pl.ds(_L, _L)]
                s = u0 * v0 + u1 * v1
                acc = jnp.where(lane == k, plsc.cumsum(s)[_L - 1], acc)
            out_v[pl.ds(j * _BCH + c0, _L)] = acc * inv
            return 0

        lax.fori_loop(0, _BCH // _L, group, 0)

    pltpu.sync_copy(out_v, out_hbm.at[pl.ds(b0, _BPW)])


_phase_b = functools.partial(
    pl.kernel,
    mesh=plsc.VectorSubcoreMesh(core_axis_name="c", subcore_axis_name="s"),
    out_type=jax.ShapeDtypeStruct((_B,), jnp.float32),
    compiler_params=pltpu.CompilerParams(
        needs_layout_passes=False, use_tc_tiling_on_sc=True),
    scratch_types=[
        pltpu.VMEM((2, _BCH, 128), jnp.float32),
        pltpu.VMEM((2, _BCH, 128), jnp.float32),
        pltpu.VMEM((2, _BCH, 128), jnp.float32),
        pltpu.VMEM((_BPW,), jnp.float32),
        pltpu.SemaphoreType.DMA,
        pltpu.SemaphoreType.DMA,
    ],
)(_b_body)


def kernel(users, pastors, trait_idx, trait_offsets, user_embed_w,
           pastor_emb_w, trait_bag_w, user_bias_w, pastor_bias_w,
           global_bias):
    del trait_offsets, user_bias_w, pastor_bias_w, global_bias
    su, sp, st = _phase_a(users, pastors, trait_idx, user_embed_w.T,
                          pastor_emb_w.T, trait_bag_w.T)
    return _phase_b(su, sp, st)
